# Initial kernel scaffold; baseline (speedup 1.0000x reference)
import jax
import jax.numpy as jnp
from jax.experimental import pallas as pl


def kernel(features, edge_index, W0, W1, W2):
    raise NotImplementedError("write your pallas kernel here")


# speedup vs baseline: 7.5304x; 7.5304x over previous
import functools

import jax
import jax.numpy as jnp
from jax import lax
from jax.experimental import pallas as pl
from jax.experimental.pallas import tpu as pltpu
from jax.experimental.pallas import tpu_sc as plsc

N = 10000
E = 160000
NUM_CORES = 2
NUM_SUBCORES = 16
NW = NUM_CORES * NUM_SUBCORES
EPW = E // NW
CHUNK = 125
NCHUNK = EPW // CHUNK
ROWS_PER_TILE = N // NUM_SUBCORES


@functools.cache
def _make_agg(H: int):
    mesh = plsc.VectorSubcoreMesh(core_axis_name="c", subcore_axis_name="s")

    @functools.partial(
        pl.kernel,
        out_type=jax.ShapeDtypeStruct((NUM_CORES, N, H), jnp.float32),
        mesh=mesh,
        compiler_params=pltpu.CompilerParams(use_tc_tiling_on_sc=False),
        scratch_types=[
            pltpu.VMEM((NCHUNK, CHUNK), jnp.int32),
            pltpu.VMEM((NCHUNK, CHUNK), jnp.int32),
            pltpu.VMEM((CHUNK, H), jnp.float32),
            pltpu.VMEM_SHARED((N, H), jnp.float32),
            pltpu.SemaphoreType.DMA,
        ],
    )
    def agg(hw_hbm, src_hbm, dst_hbm, zeros_hbm, out_hbm,
            src_v, dst_v, gbuf, acc, sem):
        cid = lax.axis_index("c")
        sid = lax.axis_index("s")
        wid = cid * NUM_SUBCORES + sid
        base = sid * ROWS_PER_TILE
        pltpu.sync_copy(zeros_hbm, acc.at[pl.ds(base, ROWS_PER_TILE)])
        pltpu.sync_copy(src_hbm.at[wid], src_v)
        pltpu.sync_copy(dst_hbm.at[wid], dst_v)
        plsc.subcore_barrier()

        def body(j, carry):
            pltpu.async_copy(hw_hbm.at[dst_v.at[j]], gbuf, sem).wait()
            pltpu.sync_copy(gbuf, acc.at[src_v.at[j]], add=True)
            return carry

        lax.fori_loop(0, NCHUNK, body, 0)
        plsc.subcore_barrier()
        pltpu.sync_copy(acc.at[pl.ds(base, ROWS_PER_TILE)],
                        out_hbm.at[cid, pl.ds(base, ROWS_PER_TILE)])

    return agg


_BM = 2000


def _mm_body(x_ref, w_ref, o_ref):
    o_ref[...] = jnp.dot(x_ref[...], w_ref[...],
                         preferred_element_type=jnp.float32)


def _mm(x, W):
    M, K = x.shape
    H = W.shape[1]
    return pl.pallas_call(
        _mm_body,
        grid=(M // _BM,),
        in_specs=[pl.BlockSpec((_BM, K), lambda i: (i, 0)),
                  pl.BlockSpec((K, H), lambda i: (0, 0))],
        out_specs=pl.BlockSpec((_BM, H), lambda i: (i, 0)),
        out_shape=jax.ShapeDtypeStruct((M, H), jnp.float32),
    )(x, W)


def _elu_mm_body(zp_ref, w_ref, o_ref):
    z = zp_ref[0] + zp_ref[1]
    h = jnp.where(z > 0, z, jnp.exp(z) - 1.0)
    o_ref[...] = jnp.dot(h, w_ref[...], preferred_element_type=jnp.float32)


def _elu_mm(zp, W):
    _, M, K = zp.shape
    H = W.shape[1]
    return pl.pallas_call(
        _elu_mm_body,
        grid=(M // _BM,),
        in_specs=[pl.BlockSpec((NUM_CORES, _BM, K), lambda i: (0, i, 0)),
                  pl.BlockSpec((K, H), lambda i: (0, 0))],
        out_specs=pl.BlockSpec((_BM, H), lambda i: (i, 0)),
        out_shape=jax.ShapeDtypeStruct((M, H), jnp.float32),
    )(zp, W)


def _final_body(zp_ref, o_ref):
    z = zp_ref[0] + zp_ref[1]
    h = jnp.where(z > 0, z, jnp.exp(z) - 1.0)
    m = jnp.max(h, axis=1, keepdims=True)
    lse = m + jnp.log(jnp.sum(jnp.exp(h - m), axis=1, keepdims=True))
    o_ref[...] = h - lse


def _final(zp):
    _, M, C = zp.shape
    return pl.pallas_call(
        _final_body,
        grid=(M // _BM,),
        in_specs=[pl.BlockSpec((NUM_CORES, _BM, C), lambda i: (0, i, 0))],
        out_specs=pl.BlockSpec((_BM, C), lambda i: (i, 0)),
        out_shape=jax.ShapeDtypeStruct((M, C), jnp.float32),
    )(zp)


def kernel(features, edge_index, W0, W1, W2):
    src = edge_index[0].astype(jnp.int32).reshape(NW, NCHUNK, CHUNK)
    dst = edge_index[1].astype(jnp.int32).reshape(NW, NCHUNK, CHUNK)
    z128 = jnp.zeros((ROWS_PER_TILE, 128), jnp.float32)
    z64 = jnp.zeros((ROWS_PER_TILE, 64), jnp.float32)

    hw = _mm(features, W0)
    zp = _make_agg(128)(hw, src, dst, z128)
    hw = _elu_mm(zp, W1)
    zp = _make_agg(128)(hw, src, dst, z128)
    hw = _elu_mm(zp, W2)
    zp = _make_agg(64)(hw, src, dst, z64)
    return _final(zp)


# speedup vs baseline: 9.1370x; 1.2133x over previous
import functools

import jax
import jax.numpy as jnp
from jax import lax
from jax.experimental import pallas as pl
from jax.experimental.pallas import tpu as pltpu
from jax.experimental.pallas import tpu_sc as plsc

N = 10000
E = 160000
NUM_CORES = 2
NUM_SUBCORES = 16
NW = NUM_CORES * NUM_SUBCORES
EPW = E // NW
CHUNK = 125
NCHUNK = EPW // CHUNK
ROWS_PER_TILE = N // NUM_SUBCORES


@functools.cache
def _make_agg(H: int):
    mesh = plsc.VectorSubcoreMesh(core_axis_name="c", subcore_axis_name="s")

    @functools.partial(
        pl.kernel,
        out_type=jax.ShapeDtypeStruct((NUM_CORES, N, H), jnp.float32),
        mesh=mesh,
        compiler_params=pltpu.CompilerParams(use_tc_tiling_on_sc=False),
        scratch_types=[
            pltpu.VMEM((NCHUNK, CHUNK), jnp.int32),
            pltpu.VMEM((NCHUNK, CHUNK), jnp.int32),
            pltpu.VMEM((CHUNK, H), jnp.float32),
            pltpu.VMEM((CHUNK, H), jnp.float32),
            pltpu.VMEM_SHARED((N, H), jnp.float32),
            pltpu.SemaphoreType.DMA,
            pltpu.SemaphoreType.DMA,
        ],
    )
    def agg(hw_hbm, src_hbm, dst_hbm, zeros_hbm, out_hbm,
            src_v, dst_v, gbuf0, gbuf1, acc, sem0, sem1):
        cid = lax.axis_index("c")
        sid = lax.axis_index("s")
        wid = cid * NUM_SUBCORES + sid
        base = sid * ROWS_PER_TILE
        pltpu.sync_copy(zeros_hbm, acc.at[pl.ds(base, ROWS_PER_TILE)])
        pltpu.sync_copy(src_hbm.at[wid], src_v)
        pltpu.sync_copy(dst_hbm.at[wid], dst_v)
        plsc.subcore_barrier()

        pltpu.async_copy(hw_hbm.at[dst_v.at[0]], gbuf0, sem0)

        def body(jj, carry):
            j0 = 2 * jj
            pltpu.make_async_copy(hw_hbm.at[dst_v.at[j0]], gbuf0, sem0).wait()
            pltpu.async_copy(hw_hbm.at[dst_v.at[j0 + 1]], gbuf1, sem1)
            pltpu.sync_copy(gbuf0, acc.at[src_v.at[j0]], add=True)
            pltpu.make_async_copy(hw_hbm.at[dst_v.at[j0 + 1]], gbuf1,
                                  sem1).wait()

            @pl.when(j0 + 2 < NCHUNK)
            def _():
                pltpu.async_copy(hw_hbm.at[dst_v.at[j0 + 2]], gbuf0, sem0)

            pltpu.sync_copy(gbuf1, acc.at[src_v.at[j0 + 1]], add=True)
            return carry

        lax.fori_loop(0, NCHUNK // 2, body, 0)
        plsc.subcore_barrier()
        pltpu.sync_copy(acc.at[pl.ds(base, ROWS_PER_TILE)],
                        out_hbm.at[cid, pl.ds(base, ROWS_PER_TILE)])

    return agg


_BM = 2000


def _mm_body(x_ref, w_ref, o_ref):
    o_ref[...] = jnp.dot(x_ref[...], w_ref[...],
                         preferred_element_type=jnp.float32)


def _mm(x, W):
    M, K = x.shape
    H = W.shape[1]
    return pl.pallas_call(
        _mm_body,
        grid=(M // _BM,),
        in_specs=[pl.BlockSpec((_BM, K), lambda i: (i, 0)),
                  pl.BlockSpec((K, H), lambda i: (0, 0))],
        out_specs=pl.BlockSpec((_BM, H), lambda i: (i, 0)),
        out_shape=jax.ShapeDtypeStruct((M, H), jnp.float32),
    )(x, W)


def _elu_mm_body(zp_ref, w_ref, o_ref):
    z = zp_ref[0] + zp_ref[1]
    h = jnp.where(z > 0, z, jnp.exp(z) - 1.0)
    o_ref[...] = jnp.dot(h, w_ref[...], preferred_element_type=jnp.float32)


def _elu_mm(zp, W):
    _, M, K = zp.shape
    H = W.shape[1]
    return pl.pallas_call(
        _elu_mm_body,
        grid=(M // _BM,),
        in_specs=[pl.BlockSpec((NUM_CORES, _BM, K), lambda i: (0, i, 0)),
                  pl.BlockSpec((K, H), lambda i: (0, 0))],
        out_specs=pl.BlockSpec((_BM, H), lambda i: (i, 0)),
        out_shape=jax.ShapeDtypeStruct((M, H), jnp.float32),
    )(zp, W)


def _final_body(zp_ref, o_ref):
    z = zp_ref[0] + zp_ref[1]
    h = jnp.where(z > 0, z, jnp.exp(z) - 1.0)
    m = jnp.max(h, axis=1, keepdims=True)
    lse = m + jnp.log(jnp.sum(jnp.exp(h - m), axis=1, keepdims=True))
    o_ref[...] = h - lse


def _final(zp):
    _, M, C = zp.shape
    return pl.pallas_call(
        _final_body,
        grid=(M // _BM,),
        in_specs=[pl.BlockSpec((NUM_CORES, _BM, C), lambda i: (0, i, 0))],
        out_specs=pl.BlockSpec((_BM, C), lambda i: (i, 0)),
        out_shape=jax.ShapeDtypeStruct((M, C), jnp.float32),
    )(zp)


def kernel(features, edge_index, W0, W1, W2):
    src = edge_index[0].astype(jnp.int32).reshape(NW, NCHUNK, CHUNK)
    dst = edge_index[1].astype(jnp.int32).reshape(NW, NCHUNK, CHUNK)
    z128 = jnp.zeros((ROWS_PER_TILE, 128), jnp.float32)
    z64 = jnp.zeros((ROWS_PER_TILE, 64), jnp.float32)

    hw = _mm(features, W0)
    zp = _make_agg(128)(hw, src, dst, z128)
    hw = _elu_mm(zp, W1)
    zp = _make_agg(128)(hw, src, dst, z128)
    hw = _elu_mm(zp, W2)
    zp = _make_agg(64)(hw, src, dst, z64)
    return _final(zp)


# speedup vs baseline: 9.2010x; 1.0070x over previous
import functools

import jax
import jax.numpy as jnp
from jax import lax
from jax.experimental import pallas as pl
from jax.experimental.pallas import tpu as pltpu
from jax.experimental.pallas import tpu_sc as plsc

N = 10000
E = 160000
NUM_CORES = 2
NUM_SUBCORES = 16
EPT = E // NUM_SUBCORES
CHUNK = 125
NCHUNK = EPT // CHUNK
ROWS_PER_TILE = N // NUM_SUBCORES


@functools.cache
def _make_agg(HC: int):
    mesh = plsc.VectorSubcoreMesh(core_axis_name="c", subcore_axis_name="s")

    @functools.partial(
        pl.kernel,
        out_type=jax.ShapeDtypeStruct((NUM_CORES, N, HC), jnp.float32),
        mesh=mesh,
        compiler_params=pltpu.CompilerParams(use_tc_tiling_on_sc=False),
        scratch_types=[
            pltpu.VMEM((NCHUNK, CHUNK), jnp.int32),
            pltpu.VMEM((NCHUNK, CHUNK), jnp.int32),
            [pltpu.VMEM((CHUNK, HC), jnp.float32)] * 4,
            [pltpu.SemaphoreType.DMA] * 4,
            [pltpu.SemaphoreType.DMA] * 4,
            pltpu.VMEM_SHARED((N, HC), jnp.float32),
        ],
    )
    def agg(hw_hbm, src_hbm, dst_hbm, zeros_hbm, out_hbm,
            src_v, dst_v, gb, gsem, ssem, acc):
        cid = lax.axis_index("c")
        sid = lax.axis_index("s")
        base = sid * ROWS_PER_TILE
        hw_c = hw_hbm.at[cid]
        pltpu.sync_copy(zeros_hbm, acc.at[pl.ds(base, ROWS_PER_TILE)])
        pltpu.sync_copy(src_hbm.at[sid], src_v)
        pltpu.sync_copy(dst_hbm.at[sid], dst_v)
        plsc.subcore_barrier()

        NPAIR = NCHUNK // 2

        def gather(j, s):
            return pltpu.make_async_copy(hw_c.at[dst_v.at[j]], gb[s],
                                         gsem[s])

        def scatter(j, s):
            return pltpu.make_async_copy(gb[s], acc.at[src_v.at[j]],
                                         ssem[s])

        def one_round(r, p):
            g0, g1 = 2 * p, 2 * p + 1
            s0, s1 = 2 - 2 * p, 3 - 2 * p

            @pl.when(r >= 2)
            def _():
                scatter(2 * (r - 2), g0).wait()
                scatter(2 * (r - 2) + 1, g1).wait()

            @pl.when(r < NPAIR)
            def _():
                gather(2 * r, g0).start()
                gather(2 * r + 1, g1).start()

            @pl.when(jnp.logical_and(r >= 1, r <= NPAIR))
            def _():
                gather(2 * (r - 1), s0).wait()
                scatter(2 * (r - 1), s0).start(add=True)
                gather(2 * (r - 1) + 1, s1).wait()
                scatter(2 * (r - 1) + 1, s1).start(add=True)

        def body(rr, carry):
            one_round(2 * rr, 0)
            one_round(2 * rr + 1, 1)
            return carry

        lax.fori_loop(0, NPAIR // 2 + 1, body, 0)
        plsc.subcore_barrier()
        pltpu.sync_copy(acc.at[pl.ds(base, ROWS_PER_TILE)],
                        out_hbm.at[cid, pl.ds(base, ROWS_PER_TILE)])

    return agg


_BM = 2000


def _mm_body(x_ref, w_ref, o_ref):
    res = jnp.dot(x_ref[...], w_ref[...], preferred_element_type=jnp.float32)
    hc = res.shape[1] // 2
    o_ref[0] = res[:, :hc]
    o_ref[1] = res[:, hc:]


def _mm(x, W):
    M, K = x.shape
    H = W.shape[1]
    return pl.pallas_call(
        _mm_body,
        grid=(M // _BM,),
        in_specs=[pl.BlockSpec((_BM, K), lambda i: (i, 0)),
                  pl.BlockSpec((K, H), lambda i: (0, 0))],
        out_specs=pl.BlockSpec((NUM_CORES, _BM, H // 2), lambda i: (0, i, 0)),
        out_shape=jax.ShapeDtypeStruct((NUM_CORES, M, H // 2), jnp.float32),
    )(x, W)


def _elu(z):
    return jnp.where(z > 0, z, jnp.exp(z) - 1.0)


def _elu_mm_body(z_ref, w_ref, o_ref):
    kc = z_ref.shape[2]
    h0 = _elu(z_ref[0])
    h1 = _elu(z_ref[1])
    res = (jnp.dot(h0, w_ref[:kc], preferred_element_type=jnp.float32)
           + jnp.dot(h1, w_ref[kc:], preferred_element_type=jnp.float32))
    hc = res.shape[1] // 2
    o_ref[0] = res[:, :hc]
    o_ref[1] = res[:, hc:]


def _elu_mm(z, W):
    _, M, KC = z.shape
    H = W.shape[1]
    return pl.pallas_call(
        _elu_mm_body,
        grid=(M // _BM,),
        in_specs=[pl.BlockSpec((NUM_CORES, _BM, KC), lambda i: (0, i, 0)),
                  pl.BlockSpec((2 * KC, H), lambda i: (0, 0))],
        out_specs=pl.BlockSpec((NUM_CORES, _BM, H // 2), lambda i: (0, i, 0)),
        out_shape=jax.ShapeDtypeStruct((NUM_CORES, M, H // 2), jnp.float32),
    )(z, W)


def _final_body(z_ref, o_ref):
    h = jnp.concatenate([_elu(z_ref[0]), _elu(z_ref[1])], axis=1)
    m = jnp.max(h, axis=1, keepdims=True)
    lse = m + jnp.log(jnp.sum(jnp.exp(h - m), axis=1, keepdims=True))
    o_ref[...] = h - lse


def _final(z):
    _, M, CC = z.shape
    return pl.pallas_call(
        _final_body,
        grid=(M // _BM,),
        in_specs=[pl.BlockSpec((NUM_CORES, _BM, CC), lambda i: (0, i, 0))],
        out_specs=pl.BlockSpec((_BM, 2 * CC), lambda i: (i, 0)),
        out_shape=jax.ShapeDtypeStruct((M, 2 * CC), jnp.float32),
    )(z)


def kernel(features, edge_index, W0, W1, W2):
    src = edge_index[0].astype(jnp.int32).reshape(NUM_SUBCORES, NCHUNK, CHUNK)
    dst = edge_index[1].astype(jnp.int32).reshape(NUM_SUBCORES, NCHUNK, CHUNK)
    z64 = jnp.zeros((ROWS_PER_TILE, 64), jnp.float32)
    z32 = jnp.zeros((ROWS_PER_TILE, 32), jnp.float32)

    hw = _mm(features, W0)
    zp = _make_agg(64)(hw, src, dst, z64)
    hw = _elu_mm(zp, W1)
    zp = _make_agg(64)(hw, src, dst, z64)
    hw = _elu_mm(zp, W2)
    zp = _make_agg(32)(hw, src, dst, z32)
    return _final(zp)
